# baseline (speedup 1.0000x reference)
import functools

import jax
import jax.numpy as jnp
from jax import lax
from jax.experimental import pallas as pl
from jax.experimental.pallas import tpu as pltpu
from jax.experimental.pallas import tpu_sc as plsc

S = 8192
D = 2048
NC = 2
NS = 16
L = 16
NW = NC * NS
CHUNK = S // NW
NB = CHUNK // L


def _body(inputs_hbm, mask_hbm, source_hbm, out_hbm,
          mask_v, buf0, buf1, mlist_src, mlist_pos, ulist_gidx, ulist_pos,
          sem_g0, sem_g1, sem_s0, sem_s1):
    wid = lax.axis_index("s") * NC + lax.axis_index("c")
    base = wid * CHUNK

    pltpu.sync_copy(mask_hbm, mask_v)

    def pf_body(j, acc):
        return acc + mask_v[pl.ds(j * L, L)]

    acc = lax.fori_loop(0, wid * NB, pf_body, jnp.zeros((L,), jnp.int32))
    carry = jnp.full((L,), jnp.sum(acc, axis=0), jnp.int32)

    zero16 = jnp.zeros((L,), jnp.int32)
    dummy16 = jnp.full((L,), S, jnp.int32)
    for j in range(NB):
        mlist_src[j, :] = zero16
        mlist_pos[j, :] = dummy16
        ulist_gidx[j, :] = zero16
        ulist_pos[j, :] = dummy16

    iota = lax.iota(jnp.int32, L)
    off_m = zero16
    off_u = zero16
    for j in range(NB):
        v = mask_v[pl.ds(base + j * L, L)]
        m = v > 0
        um = jnp.logical_not(m)
        cs = plsc.cumsum(v)
        pcnt = plsc.all_reduce_population_count(m)
        src_idx = jnp.maximum(carry + cs - 1, 0)
        p = base + j * L + iota
        dest = off_m + cs - 1
        plsc.store_scatter(mlist_src, [dest >> 4, dest & 15], src_idx, mask=m)
        plsc.store_scatter(mlist_pos, [dest >> 4, dest & 15], p, mask=m)
        cs_u = plsc.cumsum(1 - v)
        dest_u = off_u + cs_u - 1
        plsc.store_scatter(ulist_gidx, [dest_u >> 4, dest_u & 15], p, mask=um)
        plsc.store_scatter(ulist_pos, [dest_u >> 4, dest_u & 15], p, mask=um)
        off_m = off_m + pcnt
        off_u = off_u + (L - pcnt)
        carry = carry + pcnt

    nm = jnp.max(off_m, axis=0)
    n_mb = (nm + (L - 1)) >> 4
    n_ub = ((CHUNK - nm) + (L - 1)) >> 4

    def stream(table_hbm, gidx, pos, n):
        def gat(b, buf, sem):
            pltpu.async_copy(table_hbm.at[gidx.at[b]], buf, sem)

        def wgat(b, buf, sem):
            pltpu.make_async_copy(table_hbm.at[gidx.at[b]], buf, sem).wait()

        def sct(b, buf, sem):
            pltpu.async_copy(buf, out_hbm.at[pos.at[b]], sem).wait()

        @pl.when(n > 0)
        def _():
            gat(0, buf0, sem_g0)

            def loop_body(t, _):
                b0 = 2 * t
                b1 = b0 + 1

                @pl.when(b1 < n)
                def _():
                    gat(b1, buf1, sem_g1)
                wgat(b0, buf0, sem_g0)
                sct(b0, buf0, sem_s0)

                @pl.when(b1 < n)
                def _():
                    @pl.when(b1 + 1 < n)
                    def _():
                        gat(b1 + 1, buf0, sem_g0)
                    wgat(b1, buf1, sem_g1)
                    sct(b1, buf1, sem_s1)
                return 0

            lax.fori_loop(0, (n + 1) >> 1, loop_body, 0)

    stream(source_hbm, mlist_src, mlist_pos, n_mb)
    stream(inputs_hbm, ulist_gidx, ulist_pos, n_ub)


@functools.partial(
    pl.kernel,
    out_type=jax.ShapeDtypeStruct((S + 8, D), jnp.float32),
    mesh=plsc.VectorSubcoreMesh(core_axis_name="c", subcore_axis_name="s"),
    compiler_params=pltpu.CompilerParams(needs_layout_passes=False),
    scratch_types=[
        pltpu.VMEM((S,), jnp.int32),
        pltpu.VMEM((L, D), jnp.float32),
        pltpu.VMEM((L, D), jnp.float32),
        pltpu.VMEM((NB, L), jnp.int32),
        pltpu.VMEM((NB, L), jnp.int32),
        pltpu.VMEM((NB, L), jnp.int32),
        pltpu.VMEM((NB, L), jnp.int32),
        pltpu.SemaphoreType.DMA,
        pltpu.SemaphoreType.DMA,
        pltpu.SemaphoreType.DMA,
        pltpu.SemaphoreType.DMA,
    ],
)
def _sc_masked_scatter(inputs_hbm, mask_hbm, source_hbm, out_hbm, *scratch):
    _body(inputs_hbm, mask_hbm, source_hbm, out_hbm, *scratch)


def kernel(inputs_embeds, mask_1d, source):
    mask_i32 = mask_1d.astype(jnp.int32)
    padded = _sc_masked_scatter(inputs_embeds, mask_i32, source)
    return padded[:S]


# speedup vs baseline: 2.1690x; 2.1690x over previous
import functools

import jax
import jax.numpy as jnp
from jax import lax
from jax.experimental import pallas as pl
from jax.experimental.pallas import tpu as pltpu
from jax.experimental.pallas import tpu_sc as plsc

S = 8192
D = 2048
NC = 2
NS = 16
L = 16
NW = NC * NS
CHUNK = S // NW
NB = CHUNK // L
NBUF = 3


def _body(inputs_hbm, mask_hbm, source_hbm, out_hbm,
          mask_v, buf0, buf1, buf2, mlist_src, mlist_pos, ulist_gidx,
          ulist_pos, sem_g0, sem_g1, sem_g2, sem_s0, sem_s1, sem_s2):
    wid = lax.axis_index("s") * NC + lax.axis_index("c")
    base = wid * CHUNK

    pltpu.sync_copy(mask_hbm, mask_v)

    def pf_body(j, acc):
        a = acc + mask_v[pl.ds(j * 4 * L, L)]
        a = a + mask_v[pl.ds((j * 4 + 1) * L, L)]
        a = a + mask_v[pl.ds((j * 4 + 2) * L, L)]
        return a + mask_v[pl.ds((j * 4 + 3) * L, L)]

    acc = lax.fori_loop(0, wid * (NB // 4), pf_body,
                        jnp.zeros((L,), jnp.int32))
    carry = jnp.full((L,), jnp.sum(acc, axis=0), jnp.int32)

    iota = lax.iota(jnp.int32, L)
    zero16 = jnp.zeros((L,), jnp.int32)
    off_m = zero16
    off_u = zero16
    for j in range(NB):
        v = mask_v[pl.ds(base + j * L, L)]
        m = v > 0
        um = jnp.logical_not(m)
        cs = plsc.cumsum(v)
        pcnt = plsc.all_reduce_population_count(m)
        src_idx = jnp.maximum(carry + cs - 1, 0)
        p = base + j * L + iota
        dest = off_m + cs - 1
        plsc.store_scatter(mlist_src, [dest >> 4, dest & 15], src_idx, mask=m)
        plsc.store_scatter(mlist_pos, [dest >> 4, dest & 15], p, mask=m)
        cs_u = plsc.cumsum(1 - v)
        dest_u = off_u + cs_u - 1
        plsc.store_scatter(ulist_gidx, [dest_u >> 4, dest_u & 15], p, mask=um)
        plsc.store_scatter(ulist_pos, [dest_u >> 4, dest_u & 15], p, mask=um)
        off_m = off_m + pcnt
        off_u = off_u + (L - pcnt)
        carry = carry + pcnt

    nm = jnp.max(off_m, axis=0)
    nu = CHUNK - nm

    def tail_fix(lst_a, lst_b, n):
        @pl.when(lax.rem(n, L) != 0)
        def _():
            full = n >> 4
            tsel = lax.rem(full * L + iota, jnp.full((L,), n, jnp.int32))
            row = tsel >> 4
            col = tsel & 15
            full_b = jnp.full((L,), full, jnp.int32)
            plsc.store_scatter(lst_a, [full_b, iota],
                               plsc.load_gather(lst_a, [row, col]))
            plsc.store_scatter(lst_b, [full_b, iota],
                               plsc.load_gather(lst_b, [row, col]))

    tail_fix(mlist_src, mlist_pos, nm)
    tail_fix(ulist_gidx, ulist_pos, nu)

    n_mb = (nm + (L - 1)) >> 4
    n_ub = (nu + (L - 1)) >> 4

    bufs = (buf0, buf1, buf2)
    gsems = (sem_g0, sem_g1, sem_g2)
    ssems = (sem_s0, sem_s1, sem_s2)

    def stream(table_hbm, gidx, pos, n):
        def gat(b, k):
            pltpu.async_copy(table_hbm.at[gidx.at[b]], bufs[k], gsems[k])

        def wgat(b, k):
            pltpu.make_async_copy(table_hbm.at[gidx.at[b]], bufs[k],
                                  gsems[k]).wait()

        def sct(b, k):
            pltpu.async_copy(bufs[k], out_hbm.at[pos.at[b]], ssems[k])

        def wsct(b, k):
            pltpu.make_async_copy(bufs[k], out_hbm.at[pos.at[b]],
                                  ssems[k]).wait()

        @pl.when(n > 0)
        def _():
            gat(0, 0)

        @pl.when(n > 1)
        def _():
            gat(1, 1)

        def loop_body(t, _):
            for k in range(NBUF):
                b = NBUF * t + k

                @pl.when(b < n)
                def _():
                    @pl.when(b >= 1)
                    def _():
                        wsct(b - 1, (k + NBUF - 1) % NBUF)

                    @pl.when(b + 2 < n)
                    def _():
                        gat(b + 2, (k + 2) % NBUF)
                    wgat(b, k)
                    sct(b, k)
            return 0

        lax.fori_loop(0, (n + NBUF - 1) // NBUF, loop_body, 0)
        for k in range(NBUF):
            @pl.when((n > 0) & (lax.rem(n - 1, NBUF) == k))
            def _():
                wsct(n - 1, k)

    stream(source_hbm, mlist_src, mlist_pos, n_mb)
    stream(inputs_hbm, ulist_gidx, ulist_pos, n_ub)


@functools.partial(
    pl.kernel,
    out_type=jax.ShapeDtypeStruct((S, D), jnp.float32),
    mesh=plsc.VectorSubcoreMesh(core_axis_name="c", subcore_axis_name="s"),
    compiler_params=pltpu.CompilerParams(needs_layout_passes=False),
    scratch_types=[
        pltpu.VMEM((S,), jnp.int32),
        pltpu.VMEM((L, D), jnp.float32),
        pltpu.VMEM((L, D), jnp.float32),
        pltpu.VMEM((L, D), jnp.float32),
        pltpu.VMEM((NB, L), jnp.int32),
        pltpu.VMEM((NB, L), jnp.int32),
        pltpu.VMEM((NB, L), jnp.int32),
        pltpu.VMEM((NB, L), jnp.int32),
        pltpu.SemaphoreType.DMA,
        pltpu.SemaphoreType.DMA,
        pltpu.SemaphoreType.DMA,
        pltpu.SemaphoreType.DMA,
        pltpu.SemaphoreType.DMA,
        pltpu.SemaphoreType.DMA,
    ],
)
def _sc_masked_scatter(inputs_hbm, mask_hbm, source_hbm, out_hbm, *scratch):
    _body(inputs_hbm, mask_hbm, source_hbm, out_hbm, *scratch)


def kernel(inputs_embeds, mask_1d, source):
    mask_i32 = mask_1d.astype(jnp.int32)
    return _sc_masked_scatter(inputs_embeds, mask_i32, source)
